# baseline (speedup 1.0000x reference)
import functools

import jax
import jax.numpy as jnp
from jax import lax
from jax.experimental import pallas as pl
from jax.experimental.pallas import tpu as pltpu
from jax.experimental.pallas import tpu_sc as plsc

N = 10000
D = 128
E = 320000

NC = 2
NS = 16
NW = NC * NS
NPAD = 10240
ROWS_PER_TILE = NPAD // NS
CH = 128
EPW = E // NW
NCHUNK = 80
EPW_PAD = NCHUNK * CH


def _sc_aggregate_body(y_hbm, ei_hbm, out_hbm,
                       acc_sh, idx_a, idx_b, rows_a, rows_b,
                       sem_ga, sem_gb, sem_ia, sem_ib):
    c = lax.axis_index("c")
    s = lax.axis_index("s")
    wid = s * NC + c

    def _zero_row(i, carry):
        for j in range(D // 16):
            rows_a[i, pl.ds(j * 16, 16)] = jnp.zeros((16,), jnp.float32)
        return carry
    lax.fori_loop(0, CH, _zero_row, 0)
    for b in range(ROWS_PER_TILE // CH):
        pltpu.sync_copy(rows_a, acc_sh.at[pl.ds(s * ROWS_PER_TILE + b * CH, CH)])
    plsc.subcore_barrier()

    def _idx_start(j, ibuf, sem):
        pltpu.make_async_copy(ei_hbm.at[wid, j], ibuf, sem).start()

    def _idx_wait(j, ibuf, sem):
        pltpu.make_async_copy(ei_hbm.at[wid, j], ibuf, sem).wait()

    def _gather_start(ibuf, buf, sem):
        pltpu.make_async_copy(y_hbm.at[ibuf.at[0]], buf, sem).start()

    def _gather_wait(ibuf, buf, sem):
        pltpu.make_async_copy(y_hbm.at[ibuf.at[0]], buf, sem).wait()

    def _scatter(ibuf, buf):
        pltpu.sync_copy(buf, acc_sh.at[ibuf.at[1]], add=True)

    pltpu.sync_copy(ei_hbm.at[wid, 0], idx_a)
    _gather_start(idx_a, rows_a, sem_ga)
    _idx_start(1, idx_b, sem_ib)

    def _pair(i2, carry):
        j = i2 * 2
        _gather_wait(idx_a, rows_a, sem_ga)
        _idx_wait(j + 1, idx_b, sem_ib)
        _gather_start(idx_b, rows_b, sem_gb)
        _scatter(idx_a, rows_a)
        _idx_start(j + 2, idx_a, sem_ia)
        _gather_wait(idx_b, rows_b, sem_gb)
        _idx_wait(j + 2, idx_a, sem_ia)
        _gather_start(idx_a, rows_a, sem_ga)
        _scatter(idx_b, rows_b)
        _idx_start(j + 3, idx_b, sem_ib)
        return carry
    lax.fori_loop(0, NCHUNK // 2 - 1, _pair, 0)

    _gather_wait(idx_a, rows_a, sem_ga)
    _idx_wait(NCHUNK - 1, idx_b, sem_ib)
    _gather_start(idx_b, rows_b, sem_gb)
    _scatter(idx_a, rows_a)
    _gather_wait(idx_b, rows_b, sem_gb)
    _scatter(idx_b, rows_b)
    plsc.subcore_barrier()

    row0 = s * ROWS_PER_TILE
    pltpu.sync_copy(acc_sh.at[pl.ds(row0, ROWS_PER_TILE)],
                    out_hbm.at[c, pl.ds(row0, ROWS_PER_TILE)])


@jax.jit
def _sc_aggregate(y_pad, ei_pad):
    mesh = plsc.VectorSubcoreMesh(core_axis_name="c", subcore_axis_name="s")
    return pl.kernel(
        _sc_aggregate_body,
        out_type=jax.ShapeDtypeStruct((NC, NPAD, D), jnp.float32),
        mesh=mesh,
        scratch_types=[
            pltpu.VMEM_SHARED((NPAD, D), jnp.float32),
            pltpu.VMEM((2, CH), jnp.int32),
            pltpu.VMEM((2, CH), jnp.int32),
            pltpu.VMEM((CH, D), jnp.float32),
            pltpu.VMEM((CH, D), jnp.float32),
            pltpu.SemaphoreType.DMA,
            pltpu.SemaphoreType.DMA,
            pltpu.SemaphoreType.DMA,
            pltpu.SemaphoreType.DMA,
        ],
    )(y_pad, ei_pad)


def _tc_pre_body(h_ref, w_ref, y_ref):
    y_ref[...] = jnp.dot(h_ref[...], w_ref[...],
                         preferred_element_type=jnp.float32)


@jax.jit
def _tc_pre(h, w):
    return pl.pallas_call(
        _tc_pre_body,
        out_shape=jax.ShapeDtypeStruct((NPAD, D), jnp.float32),
    )(h, w)


def _tc_post_body(p_ref, h_ref, wroot_ref, b_ref, wnext_ref, h_out, y_out):
    agg = p_ref[0] + p_ref[1]
    pre = agg + jnp.dot(h_ref[...], wroot_ref[...],
                        preferred_element_type=jnp.float32) + b_ref[...]
    h = jax.nn.sigmoid(pre)
    h_out[...] = h
    y_out[...] = jnp.dot(h, wnext_ref[...], preferred_element_type=jnp.float32)


@jax.jit
def _tc_post(p, h_prev, w_root, b, w_next):
    return pl.pallas_call(
        _tc_post_body,
        out_shape=(jax.ShapeDtypeStruct((NPAD, D), jnp.float32),
                   jax.ShapeDtypeStruct((NPAD, D), jnp.float32)),
    )(p, h_prev, w_root, b.reshape(1, D), w_next)


def _tc_final_body(p_ref, h_ref, wroot_ref, b_ref, out_ref):
    agg = p_ref[0] + p_ref[1]
    out_ref[...] = agg + jnp.dot(h_ref[...], wroot_ref[...],
                                 preferred_element_type=jnp.float32) + b_ref[...]


@jax.jit
def _tc_final(p, h_prev, w_root, b):
    return pl.pallas_call(
        _tc_final_body,
        out_shape=jax.ShapeDtypeStruct((NPAD, D), jnp.float32),
    )(p, h_prev, w_root, b.reshape(1, D))


def kernel(x, edge_index, W_in_rel, b_in_rel, W_in_root,
           W_med_rel, b_med_rel, W_med_root,
           W_out_rel, b_out_rel, W_out_root):
    src = edge_index[0].astype(jnp.int32).reshape(NW, EPW)
    dst = edge_index[1].astype(jnp.int32).reshape(NW, EPW)
    src_pad = jnp.pad(src, ((0, 0), (0, EPW_PAD - EPW))).reshape(NW, NCHUNK, CH)
    dst_pad = jnp.pad(dst, ((0, 0), (0, EPW_PAD - EPW)),
                      constant_values=NPAD - 1).reshape(NW, NCHUNK, CH)
    ei_pad = jnp.stack([src_pad, dst_pad], axis=2)
    x_pad = jnp.pad(x, ((0, NPAD - N), (0, 0)))

    y1 = _tc_pre(x_pad, W_in_rel)
    p1 = _sc_aggregate(y1, ei_pad)
    h1, y2 = _tc_post(p1, x_pad, W_in_root, b_in_rel, W_med_rel)
    p2 = _sc_aggregate(y2, ei_pad)
    h2, y3 = _tc_post(p2, h1, W_med_root, b_med_rel, W_med_rel)
    p3 = _sc_aggregate(y3, ei_pad)
    h3, y4 = _tc_post(p3, h2, W_med_root, b_med_rel, W_out_rel)
    p4 = _sc_aggregate(y4, ei_pad)
    out_pad = _tc_final(p4, h3, W_out_root, b_out_rel)
    return out_pad[:N]


# speedup vs baseline: 1.0496x; 1.0496x over previous
import functools

import jax
import jax.numpy as jnp
from jax import lax
from jax.experimental import pallas as pl
from jax.experimental.pallas import tpu as pltpu
from jax.experimental.pallas import tpu_sc as plsc

N = 10000
D = 128
E = 320000

NC = 2
NS = 16
NW = NC * NS
NPAD = 10240
ROWS_PER_TILE = NPAD // NS
CH = 128
EPW = E // NW
NCHUNK = 80
EPW_PAD = NCHUNK * CH


def _sc_aggregate_body(y_hbm, ei_hbm, out_hbm, acc_sh,
                       s0, d0, s1, d1, s2, d2, s3, d3,
                       rows_a, rows_b,
                       gsem_a, gsem_b, ssem_a, ssem_b,
                       isem_0, isem_1, isem_2, isem_3):
    c = lax.axis_index("c")
    s = lax.axis_index("s")
    wid = s * NC + c
    row0 = s * ROWS_PER_TILE

    sbufs = [s0, s1, s2, s3]
    dbufs = [d0, d1, d2, d3]
    rows = [rows_a, rows_b]
    gsems = [gsem_a, gsem_b]
    ssems = [ssem_a, ssem_b]
    isems = [isem_0, isem_1, isem_2, isem_3]

    def _zero_row(i, carry):
        for j in range(D // 16):
            rows_a[i, pl.ds(j * 16, 16)] = jnp.zeros((16,), jnp.float32)
        return carry
    lax.fori_loop(0, CH, _zero_row, 0)
    for b in range(ROWS_PER_TILE // CH):
        pltpu.sync_copy(rows_a, acc_sh.at[pl.ds(row0 + b * CH, CH)])
    plsc.subcore_barrier()

    def _idx_start(j, t):
        pltpu.make_async_copy(ei_hbm.at[wid, j, 0], sbufs[t], isems[t]).start()
        pltpu.make_async_copy(ei_hbm.at[wid, j, 1], dbufs[t], isems[t]).start()

    def _idx_wait(j, t):
        pltpu.make_async_copy(ei_hbm.at[wid, j, 0], sbufs[t], isems[t]).wait()
        pltpu.make_async_copy(ei_hbm.at[wid, j, 1], dbufs[t], isems[t]).wait()

    def _g_start(t, r):
        pltpu.make_async_copy(y_hbm.at[sbufs[t]], rows[r], gsems[r]).start()

    def _g_wait(t, r):
        pltpu.make_async_copy(y_hbm.at[sbufs[t]], rows[r], gsems[r]).wait()

    def _s_start(t, r):
        pltpu.async_copy(rows[r], acc_sh.at[dbufs[t]], ssems[r], add=True)

    def _s_wait(t, r):
        pltpu.make_async_copy(rows[r], acc_sh.at[dbufs[t]], ssems[r]).wait()

    pltpu.sync_copy(ei_hbm.at[wid, 0, 0], sbufs[0])
    pltpu.sync_copy(ei_hbm.at[wid, 0, 1], dbufs[0])
    pltpu.sync_copy(ei_hbm.at[wid, 1, 0], sbufs[1])
    pltpu.sync_copy(ei_hbm.at[wid, 1, 1], dbufs[1])
    _g_start(0, 0)
    _g_start(1, 1)
    _idx_start(2, 2)
    _idx_start(3, 3)
    _g_wait(0, 0)
    _s_start(0, 0)

    def _chunk_body(j, t, r):
        _s_wait(t, r)
        _idx_start_j2(j, t)
        _idx_wait(j, t)
        _g_start(t, r)
        _g_wait_prev(j, t, r)
        _s_start_prev(j, t, r)

    def _idx_start_j2(j, t):
        tp = (t + 2) % 4
        pltpu.make_async_copy(ei_hbm.at[wid, j + 2, 0], sbufs[tp], isems[tp]).start()
        pltpu.make_async_copy(ei_hbm.at[wid, j + 2, 1], dbufs[tp], isems[tp]).start()

    def _idx_wait_j(j, t):
        pltpu.make_async_copy(ei_hbm.at[wid, j, 0], sbufs[t], isems[t]).wait()
        pltpu.make_async_copy(ei_hbm.at[wid, j, 1], dbufs[t], isems[t]).wait()

    def _g_wait_prev(j, t, r):
        _g_wait((t + 3) % 4, (r + 1) % 2)

    def _s_start_prev(j, t, r):
        _s_start((t + 3) % 4, (r + 1) % 2)

    def _quad(k, carry):
        j0 = k * 4 + 2
        for t_off in range(4):
            j = j0 + t_off
            t = (2 + t_off) % 4
            r = t_off % 2
            _s_wait(t, r)
            _idx_start_j2(j, t)
            _idx_wait_j(j, t)
            _g_start(t, r)
            _g_wait((t + 3) % 4, (r + 1) % 2)
            _s_start((t + 3) % 4, (r + 1) % 2)
        return carry
    lax.fori_loop(0, (NCHUNK - 4) // 4, _quad, 0)

    _s_wait(2, 0)
    _idx_wait_j(78, 2)
    _g_start(2, 0)
    _g_wait(1, 1)
    _s_start(1, 1)
    _s_wait(3, 1)
    _idx_wait_j(79, 3)
    _g_start(3, 1)
    _g_wait(2, 0)
    _s_start(2, 0)
    _g_wait(3, 1)
    _s_start(3, 1)
    _s_wait(2, 0)
    _s_wait(3, 1)
    plsc.subcore_barrier()

    pltpu.sync_copy(acc_sh.at[pl.ds(row0, ROWS_PER_TILE)],
                    out_hbm.at[c, pl.ds(row0, ROWS_PER_TILE)])


@jax.jit
def _sc_aggregate(y_pad, ei_pad):
    mesh = plsc.VectorSubcoreMesh(core_axis_name="c", subcore_axis_name="s")
    return pl.kernel(
        _sc_aggregate_body,
        out_type=jax.ShapeDtypeStruct((NC, NPAD, D), jnp.float32),
        mesh=mesh,
        scratch_types=[
            pltpu.VMEM_SHARED((NPAD, D), jnp.float32),
            pltpu.VMEM((CH,), jnp.int32),
            pltpu.VMEM((CH,), jnp.int32),
            pltpu.VMEM((CH,), jnp.int32),
            pltpu.VMEM((CH,), jnp.int32),
            pltpu.VMEM((CH,), jnp.int32),
            pltpu.VMEM((CH,), jnp.int32),
            pltpu.VMEM((CH,), jnp.int32),
            pltpu.VMEM((CH,), jnp.int32),
            pltpu.VMEM((CH, D), jnp.float32),
            pltpu.VMEM((CH, D), jnp.float32),
            pltpu.SemaphoreType.DMA,
            pltpu.SemaphoreType.DMA,
            pltpu.SemaphoreType.DMA,
            pltpu.SemaphoreType.DMA,
            pltpu.SemaphoreType.DMA,
            pltpu.SemaphoreType.DMA,
            pltpu.SemaphoreType.DMA,
            pltpu.SemaphoreType.DMA,
        ],
    )(y_pad, ei_pad)


def _tc_pre_body(h_ref, w_ref, y_ref):
    y_ref[...] = jnp.dot(h_ref[...], w_ref[...],
                         preferred_element_type=jnp.float32)


@jax.jit
def _tc_pre(h, w):
    return pl.pallas_call(
        _tc_pre_body,
        out_shape=jax.ShapeDtypeStruct((NPAD, D), jnp.float32),
    )(h, w)


def _tc_post_body(p_ref, h_ref, wroot_ref, b_ref, wnext_ref, h_out, y_out):
    agg = p_ref[0] + p_ref[1]
    pre = agg + jnp.dot(h_ref[...], wroot_ref[...],
                        preferred_element_type=jnp.float32) + b_ref[...]
    h = jax.nn.sigmoid(pre)
    h_out[...] = h
    y_out[...] = jnp.dot(h, wnext_ref[...], preferred_element_type=jnp.float32)


@jax.jit
def _tc_post(p, h_prev, w_root, b, w_next):
    return pl.pallas_call(
        _tc_post_body,
        out_shape=(jax.ShapeDtypeStruct((NPAD, D), jnp.float32),
                   jax.ShapeDtypeStruct((NPAD, D), jnp.float32)),
    )(p, h_prev, w_root, b.reshape(1, D), w_next)


def _tc_final_body(p_ref, h_ref, wroot_ref, b_ref, out_ref):
    agg = p_ref[0] + p_ref[1]
    out_ref[...] = agg + jnp.dot(h_ref[...], wroot_ref[...],
                                 preferred_element_type=jnp.float32) + b_ref[...]


@jax.jit
def _tc_final(p, h_prev, w_root, b):
    return pl.pallas_call(
        _tc_final_body,
        out_shape=jax.ShapeDtypeStruct((NPAD, D), jnp.float32),
    )(p, h_prev, w_root, b.reshape(1, D))


def kernel(x, edge_index, W_in_rel, b_in_rel, W_in_root,
           W_med_rel, b_med_rel, W_med_root,
           W_out_rel, b_out_rel, W_out_root):
    src = edge_index[0].astype(jnp.int32).reshape(NW, EPW)
    dst = edge_index[1].astype(jnp.int32).reshape(NW, EPW)
    src_pad = jnp.pad(src, ((0, 0), (0, EPW_PAD - EPW))).reshape(NW, NCHUNK, CH)
    dst_pad = jnp.pad(dst, ((0, 0), (0, EPW_PAD - EPW)),
                      constant_values=NPAD - 1).reshape(NW, NCHUNK, CH)
    ei_pad = jnp.stack([src_pad, dst_pad], axis=2)
    x_pad = jnp.pad(x, ((0, NPAD - N), (0, 0)))

    y1 = _tc_pre(x_pad, W_in_rel)
    p1 = _sc_aggregate(y1, ei_pad)
    h1, y2 = _tc_post(p1, x_pad, W_in_root, b_in_rel, W_med_rel)
    p2 = _sc_aggregate(y2, ei_pad)
    h2, y3 = _tc_post(p2, h1, W_med_root, b_med_rel, W_med_rel)
    p3 = _sc_aggregate(y3, ei_pad)
    h3, y4 = _tc_post(p3, h2, W_med_root, b_med_rel, W_out_rel)
    p4 = _sc_aggregate(y4, ei_pad)
    out_pad = _tc_final(p4, h3, W_out_root, b_out_rel)
    return out_pad[:N]
